# baseline (speedup 1.0000x reference)
import functools

import jax
import jax.numpy as jnp
from jax import lax
from jax.experimental import pallas as pl
from jax.experimental.pallas import tpu as pltpu
from jax.experimental.pallas import tpu_sc as plsc

B = 16384
V = 1000000
CTX = 10
NEG = 20
D = 64
NSCORE = NEG + 1
NC, NS, L = 2, 16, 16
NW = NC * NS
BPW = B // NW
NB = 16
NCHUNK = BPW // NB


def _sc_scores(cp, cc, tp, tc, np_, nc_, in2, out2):
  mesh = plsc.VectorSubcoreMesh(core_axis_name="c", subcore_axis_name="s")

  @functools.partial(
      pl.kernel,
      out_type=jax.ShapeDtypeStruct((B * NSCORE,), jnp.float32),
      mesh=mesh,
      scratch_types=[
          pltpu.VMEM((CTX * NB,), jnp.int32),
          pltpu.VMEM((CTX * NB,), jnp.int32),
          pltpu.VMEM((NB,), jnp.int32),
          pltpu.VMEM((NB,), jnp.int32),
          pltpu.VMEM((NEG * NB,), jnp.int32),
          pltpu.VMEM((NEG * NB,), jnp.int32),
          pltpu.VMEM((CTX * NB, 2 * D), jnp.float32),
          pltpu.VMEM((NB, 2 * D), jnp.float32),
          pltpu.VMEM((NEG * NB, 2 * D), jnp.float32),
          pltpu.VMEM((NSCORE * NB,), jnp.float32),
          pltpu.SemaphoreType.DMA,
          pltpu.SemaphoreType.DMA,
      ],
      compiler_params=pltpu.CompilerParams(
          needs_layout_passes=False, use_tc_tiling_on_sc=False),
  )
  def k(cp_hbm, cc_hbm, tp_hbm, tc_hbm, np_hbm, nc_hbm, in2_hbm, out2_hbm,
        out_hbm, ctx_pi, ctx_cb, tgt_pi, tgt_cb, neg_pi, neg_cb,
        ctx_rows, tgt_rows, neg_rows, out_stage, sem_i, sem_r):
    wid = lax.axis_index("s") * NC + lax.axis_index("c")
    iota = lax.iota(jnp.int32, L)
    rows10 = iota * CTX
    rows20 = iota * NEG

    def chunk(g, carry):
      co = (wid * BPW + g * NB) * CTX
      to = wid * BPW + g * NB
      no = (wid * BPW + g * NB) * NEG
      c1 = pltpu.async_copy(cp_hbm.at[pl.ds(co, CTX * NB)], ctx_pi, sem_i)
      c2 = pltpu.async_copy(cc_hbm.at[pl.ds(co, CTX * NB)], ctx_cb, sem_i)
      c3 = pltpu.async_copy(tp_hbm.at[pl.ds(to, NB)], tgt_pi, sem_i)
      c4 = pltpu.async_copy(tc_hbm.at[pl.ds(to, NB)], tgt_cb, sem_i)
      c5 = pltpu.async_copy(np_hbm.at[pl.ds(no, NEG * NB)], neg_pi, sem_i)
      c6 = pltpu.async_copy(nc_hbm.at[pl.ds(no, NEG * NB)], neg_cb, sem_i)
      c1.wait()
      c2.wait()
      c3.wait()
      c4.wait()
      c5.wait()
      c6.wait()

      g1 = pltpu.async_copy(in2_hbm.at[ctx_pi.at[pl.ds(0, 128)]],
                            ctx_rows.at[pl.ds(0, 128), :], sem_r)
      g2 = pltpu.async_copy(in2_hbm.at[ctx_pi.at[pl.ds(128, 32)]],
                            ctx_rows.at[pl.ds(128, 32), :], sem_r)
      g3 = pltpu.async_copy(out2_hbm.at[tgt_pi], tgt_rows, sem_r)
      g4 = pltpu.async_copy(out2_hbm.at[neg_pi.at[pl.ds(0, 128)]],
                            neg_rows.at[pl.ds(0, 128), :], sem_r)
      g5 = pltpu.async_copy(out2_hbm.at[neg_pi.at[pl.ds(128, 128)]],
                            neg_rows.at[pl.ds(128, 128), :], sem_r)
      g6 = pltpu.async_copy(out2_hbm.at[neg_pi.at[pl.ds(256, 64)]],
                            neg_rows.at[pl.ds(256, 64), :], sem_r)

      ccol = [plsc.load_gather(ctx_cb, [rows10 + j]) for j in range(CTX)]
      tcol = plsc.load_gather(tgt_cb, [iota])
      ncol = [plsc.load_gather(neg_cb, [rows20 + n]) for n in range(NEG)]

      g1.wait()
      g2.wait()
      g3.wait()
      g4.wait()
      g5.wait()
      g6.wait()

      def dstep(d, acc):
        dv = (iota + d) & (D - 1)
        c = plsc.load_gather(ctx_rows, [rows10, ccol[0] + dv])
        for j in range(1, CTX):
          c = c + plsc.load_gather(ctx_rows, [rows10 + j, ccol[j] + dv])
        t = plsc.load_gather(tgt_rows, [iota, tcol + dv])
        pos = acc[0] + c * t
        negs = [
            acc[1 + n]
            + c * plsc.load_gather(neg_rows, [rows20 + n, ncol[n] + dv])
            for n in range(NEG)
        ]
        return (pos, *negs)

      zero = jnp.zeros((L,), jnp.float32)
      acc = lax.fori_loop(0, D, dstep, (zero,) * NSCORE)
      scale = jnp.float32(1.0 / CTX)
      oidx = iota * NSCORE
      plsc.store_scatter(out_stage, [oidx], acc[0] * scale)
      for n in range(NEG):
        plsc.store_scatter(out_stage, [oidx + (1 + n)], acc[1 + n] * (-scale))
      pltpu.sync_copy(
          out_stage,
          out_hbm.at[pl.ds((wid * BPW + g * NB) * NSCORE, NSCORE * NB)])
      return carry

    lax.fori_loop(0, NCHUNK, chunk, 0)

  return k(cp, cc, tp, tc, np_, nc_, in2, out2)


def _tc_loss(scores2d):
  def body(x_ref, o_ref):
    ls = jax.nn.log_sigmoid(x_ref[...])
    o_ref[0, 0] = -jnp.sum(ls) / jnp.float32(B)

  return pl.pallas_call(
      body,
      out_shape=jax.ShapeDtypeStruct((1, 1), jnp.float32),
      out_specs=pl.BlockSpec(memory_space=pltpu.SMEM),
  )(scores2d)


def kernel(context_words, target_words, negative_words, input_embeddings,
           output_embeddings):
  ctx_flat = context_words.reshape(-1).astype(jnp.int32)
  neg_flat = negative_words.reshape(-1).astype(jnp.int32)
  tgt = target_words.astype(jnp.int32)
  cp, cc = ctx_flat >> 1, (ctx_flat & 1) << 6
  tp, tc = tgt >> 1, (tgt & 1) << 6
  np_, nc_ = neg_flat >> 1, (neg_flat & 1) << 6
  in2 = input_embeddings.reshape(V // 2, 2 * D)
  out2 = output_embeddings.reshape(V // 2, 2 * D)
  scores = _sc_scores(cp, cc, tp, tc, np_, nc_, in2, out2)
  loss = _tc_loss(scores.reshape(B * NSCORE // 128, 128))
  return loss[0, 0]


# speedup vs baseline: 1.1653x; 1.1653x over previous
import functools

import jax
import jax.numpy as jnp
from jax import lax
from jax.experimental import pallas as pl
from jax.experimental.pallas import tpu as pltpu
from jax.experimental.pallas import tpu_sc as plsc

B = 16384
CTX = 10
NEG = 20
D = 64
NSCORE = NEG + 1
NC, NS, L = 2, 16, 16
NW = NC * NS
BPW = B // NW
NB = 16
NCHUNK = BPW // NB
NBUF = 2


def _sc_scores(ctx_flat, tgt, neg_flat, in_emb, out_emb):
  mesh = plsc.VectorSubcoreMesh(core_axis_name="c", subcore_axis_name="s")

  @functools.partial(
      pl.kernel,
      out_type=jax.ShapeDtypeStruct((B * NSCORE,), jnp.float32),
      mesh=mesh,
      scratch_types=[
          pltpu.VMEM((BPW * CTX,), jnp.int32),
          pltpu.VMEM((BPW,), jnp.int32),
          pltpu.VMEM((BPW * NEG,), jnp.int32),
          pltpu.VMEM((NBUF, CTX * NB, D), jnp.float32),
          pltpu.VMEM((NBUF, NB, D), jnp.float32),
          pltpu.VMEM((NBUF, NEG * NB, D), jnp.float32),
          pltpu.VMEM((BPW * NSCORE,), jnp.float32),
          pltpu.SemaphoreType.DMA,
          pltpu.SemaphoreType.DMA,
          pltpu.SemaphoreType.DMA,
      ],
      compiler_params=pltpu.CompilerParams(
          needs_layout_passes=False, use_tc_tiling_on_sc=False),
  )
  def k(ctx_hbm, tgt_hbm, neg_hbm, ine_hbm, oute_hbm, out_hbm,
        ctx_idx, tgt_idx, neg_idx, ctx_rows, tgt_rows, neg_rows,
        out_all, sem_i, sem_r0, sem_r1):
    wid = lax.axis_index("s") * NC + lax.axis_index("c")
    iota = lax.iota(jnp.int32, L)
    rows10 = iota * CTX
    rows20 = iota * NEG
    sems = (sem_r0, sem_r1)

    def fire(g, b):
      sem = sems[b]
      co = g * CTX * NB
      no = g * NEG * NB
      pltpu.async_copy(ine_hbm.at[ctx_idx.at[pl.ds(co, 128)]],
                       ctx_rows.at[b, pl.ds(0, 128), :], sem)
      pltpu.async_copy(ine_hbm.at[ctx_idx.at[pl.ds(co + 128, 32)]],
                       ctx_rows.at[b, pl.ds(128, 32), :], sem)
      pltpu.async_copy(oute_hbm.at[tgt_idx.at[pl.ds(g * NB, NB)]],
                       tgt_rows.at[b], sem)
      pltpu.async_copy(oute_hbm.at[neg_idx.at[pl.ds(no, 128)]],
                       neg_rows.at[b, pl.ds(0, 128), :], sem)
      pltpu.async_copy(oute_hbm.at[neg_idx.at[pl.ds(no + 128, 128)]],
                       neg_rows.at[b, pl.ds(128, 128), :], sem)
      pltpu.async_copy(oute_hbm.at[neg_idx.at[pl.ds(no + 256, 64)]],
                       neg_rows.at[b, pl.ds(256, 64), :], sem)

    def drain(b):
      sem = sems[b]
      pltpu.make_async_copy(ine_hbm.at[ctx_idx.at[pl.ds(0, 128)]],
                            ctx_rows.at[b, pl.ds(0, 128), :], sem).wait()
      pltpu.make_async_copy(ine_hbm.at[ctx_idx.at[pl.ds(0, 32)]],
                            ctx_rows.at[b, pl.ds(128, 32), :], sem).wait()
      pltpu.make_async_copy(oute_hbm.at[tgt_idx.at[pl.ds(0, NB)]],
                            tgt_rows.at[b], sem).wait()
      pltpu.make_async_copy(oute_hbm.at[neg_idx.at[pl.ds(0, 128)]],
                            neg_rows.at[b, pl.ds(0, 128), :], sem).wait()
      pltpu.make_async_copy(oute_hbm.at[neg_idx.at[pl.ds(0, 128)]],
                            neg_rows.at[b, pl.ds(128, 128), :], sem).wait()
      pltpu.make_async_copy(oute_hbm.at[neg_idx.at[pl.ds(0, 64)]],
                            neg_rows.at[b, pl.ds(256, 64), :], sem).wait()

    def compute(g, b):
      def dstep(d, acc):
        dv = (iota + d) & (D - 1)
        c = plsc.load_gather(ctx_rows.at[b], [rows10, dv])
        for j in range(1, CTX):
          c = c + plsc.load_gather(ctx_rows.at[b], [rows10 + j, dv])
        t = plsc.load_gather(tgt_rows.at[b], [iota, dv])
        pos = acc[0] + c * t
        negs = [
            acc[1 + n] + c * plsc.load_gather(neg_rows.at[b], [rows20 + n, dv])
            for n in range(NEG)
        ]
        return (pos, *negs)

      zero = jnp.zeros((L,), jnp.float32)
      acc = lax.fori_loop(0, D, dstep, (zero,) * NSCORE)
      scale = jnp.float32(1.0 / CTX)
      oidx = (g * NB + iota) * NSCORE
      plsc.store_scatter(out_all, [oidx], acc[0] * scale)
      for n in range(NEG):
        plsc.store_scatter(out_all, [oidx + (1 + n)], acc[1 + n] * (-scale))

    ci = pltpu.async_copy(
        ctx_hbm.at[pl.ds(wid * BPW * CTX, BPW * CTX)], ctx_idx, sem_i)
    ti = pltpu.async_copy(tgt_hbm.at[pl.ds(wid * BPW, BPW)], tgt_idx, sem_i)
    ni = pltpu.async_copy(
        neg_hbm.at[pl.ds(wid * BPW * NEG, BPW * NEG)], neg_idx, sem_i)
    ci.wait()
    ti.wait()
    ni.wait()

    for b in range(NBUF):
      fire(b, b)

    def pair(p, carry):
      g = p * NBUF
      for b in range(NBUF):
        drain(b)
        compute(g + b, b)
        @pl.when(g + b + NBUF < NCHUNK)
        def _():
          fire(g + b + NBUF, b)
      return carry

    lax.fori_loop(0, NCHUNK // NBUF, pair, 0)

    pltpu.sync_copy(
        out_all, out_hbm.at[pl.ds(wid * BPW * NSCORE, BPW * NSCORE)])

  return k(ctx_flat, tgt, neg_flat, in_emb, out_emb)


def _tc_loss(scores2d):
  def body(x_ref, o_ref):
    ls = jax.nn.log_sigmoid(x_ref[...])
    o_ref[0, 0] = -jnp.sum(ls) / jnp.float32(B)

  return pl.pallas_call(
      body,
      out_shape=jax.ShapeDtypeStruct((1, 1), jnp.float32),
      out_specs=pl.BlockSpec(memory_space=pltpu.SMEM),
  )(scores2d)


def kernel(context_words, target_words, negative_words, input_embeddings,
           output_embeddings):
  ctx_flat = context_words.reshape(-1).astype(jnp.int32)
  neg_flat = negative_words.reshape(-1).astype(jnp.int32)
  tgt = target_words.astype(jnp.int32)
  scores = _sc_scores(ctx_flat, tgt, neg_flat, input_embeddings,
                      output_embeddings)
  loss = _tc_loss(scores.reshape(B * NSCORE // 128, 128))
  return loss[0, 0]
